# baseline (speedup 1.0000x reference)
import functools

import jax
import jax.numpy as jnp
from jax import lax
from jax.experimental import pallas as pl
from jax.experimental.pallas import tpu as pltpu
from jax.experimental.pallas import tpu_sc as plsc

_CHUNK = 64
_G = 4
_LANES = 16


def _sc_lookup(tokens, table_flat, B, C, T, V, D):
    n_workers = 32
    halves = n_workers // B
    t_span = T // halves
    n_chunks = t_span // _CHUNK
    n_grp = C // _G
    n_it = n_grp * n_chunks

    mesh = plsc.VectorSubcoreMesh(core_axis_name="c", subcore_axis_name="s")

    @functools.partial(
        pl.kernel,
        mesh=mesh,
        out_type=jax.ShapeDtypeStruct((B, T, C, D), jnp.float32),
        scratch_types=[
            pltpu.VMEM((C, t_span), jnp.int32),
            pltpu.VMEM((2, _G, _CHUNK, D), jnp.float32),
            pltpu.VMEM((2, _CHUNK, _G, D), jnp.float32),
            pltpu.SemaphoreType.DMA,
            pltpu.SemaphoreType.DMA,
            pltpu.SemaphoreType.DMA,
            pltpu.SemaphoreType.DMA,
        ],
        compiler_params=pltpu.CompilerParams(use_tc_tiling_on_sc=False),
    )
    def body(tokens_hbm, table_hbm, out_hbm, idx_v, rows_v, grp_v, g0, g1, w0, w1):
        wid = lax.axis_index("s") * 2 + lax.axis_index("c")
        b = wid // halves
        t_base = (wid % halves) * t_span
        g_sem = (g0, g1)
        w_sem = (w0, w1)

        pltpu.sync_copy(tokens_hbm.at[b, :, pl.ds(t_base, t_span)], idx_v)

        def add_off(i, carry):
            off = i * V
            for j in range(t_span // _LANES):
                sl = pl.ds(j * _LANES, _LANES)
                idx_v[i, sl] = idx_v[i, sl] + off
            return carry

        lax.fori_loop(1, C, add_off, 0)

        def gathers(it, p):
            cb0 = (it % n_grp) * _G
            k = (it // n_grp) * _CHUNK
            return [
                pltpu.make_async_copy(
                    table_hbm.at[idx_v.at[cb0 + c, pl.ds(k, _CHUNK)]],
                    rows_v.at[p, c],
                    g_sem[p],
                )
                for c in range(_G)
            ]

        def reorder(p):
            def rt(t, carry):
                for c in range(_G):
                    for j in range(D // _LANES):
                        sl = pl.ds(j * _LANES, _LANES)
                        grp_v[p, t, c, sl] = rows_v[p, c, t, sl]
                return carry

            lax.fori_loop(0, _CHUNK, rt, 0)

        def write(it, p):
            cb0 = (it % n_grp) * _G
            t0 = t_base + (it // n_grp) * _CHUNK
            return pltpu.make_async_copy(
                grp_v.at[p],
                out_hbm.at[b, pl.ds(t0, _CHUNK), pl.ds(cb0, _G)],
                w_sem[p],
            )

        def stage(it, p, q):
            @pl.when(it + 1 < n_it)
            def _():
                for d in gathers(it + 1, q):
                    d.start()

            @pl.when(it >= 2)
            def _():
                write(it - 2, p).wait()

            for d in gathers(it, p):
                d.wait()
            reorder(p)
            write(it, p).start()

        for d in gathers(0, 0):
            d.start()

        def pair(it2, carry):
            stage(2 * it2, 0, 1)
            stage(2 * it2 + 1, 1, 0)
            return carry

        lax.fori_loop(0, n_it // 2, pair, 0)
        write(n_it - 2, 0).wait()
        write(n_it - 1, 1).wait()

    return body(tokens, table_flat)


def kernel(tokens, tables):
    B, C, T = tokens.shape
    C2, V, D = tables.shape
    assert C == C2
    table_flat = tables.reshape(C * V, D)
    out = _sc_lookup(tokens.astype(jnp.int32), table_flat, B, C, T, V, D)
    return out.reshape(B, T, C * D)


# speedup vs baseline: 2.5449x; 2.5449x over previous
import functools

import jax
import jax.numpy as jnp
from jax import lax
from jax.experimental import pallas as pl
from jax.experimental.pallas import tpu as pltpu
from jax.experimental.pallas import tpu_sc as plsc

_ROWS = 128
_LANES = 16


def _sc_lookup(tokens, table_flat, B, C, T, V, D):
    t_half = T // 2
    n_flat = t_half * C
    n_it = n_flat // _ROWS
    tok_per_it = _ROWS // C

    mesh = plsc.VectorSubcoreMesh(core_axis_name="c", subcore_axis_name="s")

    @functools.partial(
        pl.kernel,
        mesh=mesh,
        out_type=jax.ShapeDtypeStruct((B, 2, n_flat, D), jnp.float32),
        scratch_types=[
            pltpu.VMEM((C, t_half), jnp.int32),
            pltpu.VMEM((n_flat,), jnp.int32),
            pltpu.VMEM((4, _ROWS, D), jnp.float32),
            pltpu.SemaphoreType.DMA,
            pltpu.SemaphoreType.DMA,
            pltpu.SemaphoreType.DMA,
            pltpu.SemaphoreType.DMA,
            pltpu.SemaphoreType.DMA,
            pltpu.SemaphoreType.DMA,
            pltpu.SemaphoreType.DMA,
            pltpu.SemaphoreType.DMA,
        ],
        compiler_params=pltpu.CompilerParams(
            use_tc_tiling_on_sc=False, needs_layout_passes=False
        ),
    )
    def body(
        tokens_hbm, table_hbm, out_hbm, raw_v, idx_v, rows_v,
        g0, g1, g2, g3, w0, w1, w2, w3,
    ):
        b = lax.axis_index("s")
        half = lax.axis_index("c")
        g_sem = (g0, g1, g2, g3)
        w_sem = (w0, w1, w2, w3)

        pltpu.sync_copy(tokens_hbm.at[b, :, pl.ds(half * t_half, t_half)], raw_v)
        lane = lax.broadcasted_iota(jnp.int32, (_LANES,), 0)

        n_tb = t_half // _LANES
        lane_c = lane * C

        def transpose_fold(n, carry):
            i = n // n_tb
            t0 = (n % n_tb) * _LANES
            vals = raw_v[i, pl.ds(t0, _LANES)] + i * V
            plsc.store_scatter(idx_v, [lane_c + (t0 * C + i)], vals)
            return carry

        lax.fori_loop(0, C * n_tb, transpose_fold, 0)

        def gather(it, p):
            return pltpu.make_async_copy(
                table_hbm.at[idx_v.at[pl.ds(it * _ROWS, _ROWS)]],
                rows_v.at[p],
                g_sem[p],
            )

        def write(it, p):
            return pltpu.make_async_copy(
                rows_v.at[p],
                out_hbm.at[b, half, pl.ds(it * _ROWS, _ROWS)],
                w_sem[p],
            )

        def stage(it, p, q):
            @pl.when(it >= 1)
            def _():
                write(it - 1, q).wait()

            @pl.when(it + 3 < n_it)
            def _():
                gather(it + 3, q).start()

            gather(it, p).wait()
            write(it, p).start()

        for k in range(3):
            gather(k, k).start()

        def quad(it4, carry):
            for r in range(4):
                stage(4 * it4 + r, r, (r + 3) % 4)
            return carry

        lax.fori_loop(0, n_it // 4, quad, 0)
        write(n_it - 1, 3).wait()

    return body(tokens, table_flat)


def kernel(tokens, tables):
    B, C, T = tokens.shape
    C2, V, D = tables.shape
    assert C == C2
    table_flat = tables.reshape(C * V, D)
    out = _sc_lookup(tokens.astype(jnp.int32), table_flat, B, C, T, V, D)
    return out.reshape(B, T, C * D)
